# baseline (speedup 1.0000x reference)
import functools

import jax
import jax.numpy as jnp
from jax import lax
from jax.experimental import pallas as pl
from jax.experimental.pallas import tpu as pltpu
from jax.experimental.pallas import tpu_sc as plsc

N = 100000
C = 128
BATCH = 64
OY, OX = 32, 32
NOUT = BATCH * OY * OX
NC, NS = 2, 16
NW = NC * NS

PTS_W = N // NW
KPAD = 3136
CO_W = 3 * PTS_W
CBUF = 9392
SENT = 1 << 20
NKEYS = NW * KPAD

NREG = 128
ROWS = NOUT // NREG
SP = 129
NPASS = NREG // NW
G = 128
SEGCH = 3072
SEGBUF = SEGCH + G
PKS = 103104

_mesh = plsc.VectorSubcoreMesh(
    core_axis_name="c", subcore_axis_name="s", num_cores=NC, num_subcores=NS)
_sc_params = pltpu.CompilerParams(needs_layout_passes=False)


def _wid():
  return lax.axis_index("s") * NC + lax.axis_index("c")


@functools.partial(
    pl.kernel,
    out_type=[jax.ShapeDtypeStruct((NKEYS,), jnp.int32),
              jax.ShapeDtypeStruct((NW * NREG,), jnp.int32)],
    mesh=_mesh,
    scratch_types=[
        pltpu.VMEM((CBUF,), jnp.int32),
        pltpu.VMEM((KPAD,), jnp.int32),
        pltpu.VMEM((NREG * 16,), jnp.int32),
        pltpu.VMEM((NREG,), jnp.int32),
    ],
    compiler_params=_sc_params,
)
def _keys_kernel(coors_hbm, keys_hbm, hist_hbm, cbuf, kbuf, hsub, hrow):
  w = _wid()
  start = w * CO_W
  astart = (start // 8) * 8
  r = start - astart
  pltpu.sync_copy(coors_hbm.at[pl.ds(astart, CBUF)], cbuf)
  lanes = jnp.arange(16, dtype=jnp.int32)
  zeros = jnp.zeros((16,), jnp.int32)
  ones = jnp.ones((16,), jnp.int32)

  def hz(i, _):
    hsub[pl.ds(i * 16, 16)] = zeros
    return 0

  lax.fori_loop(0, NREG, hz, 0)

  def body(i, _):
    p = i * 16
    valid = (p + lanes) < PTS_W
    base = r + 3 * p + lanes * 3
    idx = jnp.where(valid, base, 0)
    bcol = plsc.load_gather(cbuf, [idx])
    ycol = plsc.load_gather(cbuf, [idx + 1])
    xcol = plsc.load_gather(cbuf, [idx + 2])
    key = bcol * (OY * OX) + (ycol >> 1) * OX + (xcol >> 1)
    kbuf[pl.ds(p, 16)] = jnp.where(valid, key, SENT)
    reg = jnp.where(valid, key >> 9, 0)
    plsc.addupdate_scatter(hsub, [reg * 16 + lanes], ones, mask=valid)
    return 0

  lax.fori_loop(0, KPAD // 16, body, 0)
  for rg in range(NREG // 16):
    acc16 = zeros
    for rr in range(16):
      s = jnp.sum(hsub[pl.ds((rg * 16 + rr) * 16, 16)], axis=0)
      acc16 = jnp.where(lanes == rr, s, acc16)
    hrow[pl.ds(rg * 16, 16)] = acc16
  pltpu.sync_copy(kbuf, keys_hbm.at[pl.ds(w * KPAD, KPAD)])
  pltpu.sync_copy(hrow, hist_hbm.at[pl.ds(w * NREG, NREG)])


@functools.partial(
    pl.kernel,
    out_type=[jax.ShapeDtypeStruct((PKS,), jnp.int32),
              jax.ShapeDtypeStruct((160,), jnp.int32)],
    mesh=_mesh,
    scratch_types=[
        pltpu.VMEM((KPAD,), jnp.int32),
        pltpu.VMEM((NW * NREG,), jnp.int32),
        pltpu.VMEM((NREG + 16,), jnp.int32),
        pltpu.VMEM((160,), jnp.int32),
        pltpu.VMEM((KPAD,), jnp.int32),
        pltpu.VMEM((3216,), jnp.int32),
        pltpu.VMEM((3216,), jnp.int32),
        pltpu.VMEM((8, G), jnp.int32),
        pltpu.VMEM((8, G), jnp.int32),
        pltpu.VMEM((2048,), jnp.int32),
        pltpu.VMEM((48,), jnp.int32),
    ] + [pltpu.SemaphoreType.DMA] * 8,
    compiler_params=_sc_params,
)
def _permute_kernel(keys_hbm, hist_hbm, pks_hbm, gst_hbm, kbuf, hbuf, offs,
                    gsb, regb, ent, dst, dstb, entb, zbuf, shbuf, *psems):
  w = _wid()
  pltpu.sync_copy(keys_hbm.at[pl.ds(w * KPAD, KPAD)], kbuf)
  pltpu.sync_copy(hist_hbm, hbuf)
  lanes = jnp.arange(16, dtype=jnp.int32)
  zeros = jnp.zeros((16,), jnp.int32)
  lane0 = lanes == 0
  wv = jnp.full((16,), w, jnp.int32)

  carry_gs = jnp.int32(0)
  for rg in range(NREG // 16):
    def accw(wp, c, rg=rg):
      tot, mine = c
      row = hbuf[pl.ds(wp * NREG + rg * 16, 16)]
      tot = tot + row
      mine = mine + jnp.where(jnp.full((16,), wp, jnp.int32) < wv, row, 0)
      return (tot, mine)

    tot, mine = lax.fori_loop(0, NW, accw, (zeros, zeros))
    ex = plsc.cumsum(tot) - tot
    gs_vec = ex + carry_gs
    offs[pl.ds(rg * 16, 16)] = gs_vec + mine
    gsb[pl.ds(rg * 16, 16)] = gs_vec
    carry_gs = carry_gs + jnp.sum(tot, axis=0)
  gsb[pl.ds(NREG, 16)] = jnp.full((16,), N, jnp.int32)
  gsb[pl.ds(NREG + 16, 16)] = jnp.full((16,), N, jnp.int32)

  @pl.when(w == 0)
  def _():
    pltpu.sync_copy(gsb, gst_hbm)

  def zb(i, _):
    zbuf[pl.ds(i * 16, 16)] = zeros
    return 0

  lax.fori_loop(0, 128, zb, 0)

  @pl.when(w == 1)
  def _():
    pltpu.sync_copy(zbuf, pks_hbm.at[pl.ds(N, 2048)])
    pltpu.sync_copy(zbuf.at[pl.ds(0, PKS - N - 2048)],
                    pks_hbm.at[pl.ds(N + 2048, PKS - N - 2048)])

  del lane0
  def mkent(i, _):
    p = i * 16
    key = kbuf[pl.ds(p, 16)]
    ent[pl.ds(p, 16)] = ((key & 511) << 17) | (w * PTS_W + p + lanes)
    regb[pl.ds(p, 16)] = jnp.minimum(key >> 9, NREG)
    return 0

  lax.fori_loop(0, KPAD // 16, mkent, 0)

  big = jnp.full((16,), 0x7FFFFFFF, jnp.int32)
  negone = jnp.full((16,), -1, jnp.int32)

  def pj16(i, _):
    p = i * 16
    reg = regb[pl.ds(p, 16)]
    srt_reg, srt_lane = plsc.sort_key_val(reg, lanes)
    shbuf[pl.ds(0, 16)] = negone
    shbuf[pl.ds(1, 16)] = srt_reg
    prev = shbuf[pl.ds(0, 16)]
    shbuf[pl.ds(16, 16)] = big
    shbuf[pl.ds(0, 16)] = srt_reg
    nxt = shbuf[pl.ds(1, 16)]
    newrun = srt_reg != prev
    runend = srt_reg != nxt
    runstart = plsc.cummax(jnp.where(newrun, lanes, 0))
    rank = lanes - runstart
    base = plsc.load_gather(offs, [srt_reg])
    dstv = base + rank
    plsc.store_scatter(offs, [srt_reg], dstv + 1, mask=runend)
    plsc.store_scatter(dst, [srt_lane + p], dstv)
    return 0

  lax.fori_loop(0, KPAD // 16, pj16, 0)

  le = jnp.full((16,), ent[pl.ds(PTS_W - 1, 16)][0], jnp.int32)
  ld = jnp.full((16,), dst[pl.ds(PTS_W - 1, 16)][0], jnp.int32)
  for t in range(5):
    ent[pl.ds(PTS_W + t * 16, 16)] = le
    dst[pl.ds(PTS_W + t * 16, 16)] = ld

  nb = (PTS_W + G - 1) // G

  for b in range(nb):
    k = b % 8
    if b >= 8:
      pltpu.make_async_copy(entb.at[k], pks_hbm.at[dstb.at[k]],
                            psems[k]).wait()
    for u in range(G // 16):
      dstb[k, pl.ds(u * 16, 16)] = dst[pl.ds(b * G + u * 16, 16)]
      entb[k, pl.ds(u * 16, 16)] = ent[pl.ds(b * G + u * 16, 16)]
    pltpu.async_copy(entb.at[k], pks_hbm.at[dstb.at[k]], psems[k])
  for k in range(8):
    pltpu.make_async_copy(entb.at[k], pks_hbm.at[dstb.at[k]],
                          psems[k]).wait()


@functools.partial(
    pl.kernel,
    out_type=jax.ShapeDtypeStruct((BATCH, C, OY * OX), jnp.float32),
    mesh=_mesh,
    scratch_types=[
        pltpu.VMEM((ROWS * SP,), jnp.float32),
        pltpu.VMEM((SEGBUF,), jnp.int32),
        pltpu.VMEM((SEGBUF,), jnp.int32),
        pltpu.VMEM((G,), jnp.int32),
        pltpu.VMEM((G,), jnp.int32),
        pltpu.VMEM((G, C), jnp.float32),
        pltpu.VMEM((G, C), jnp.float32),
        pltpu.VMEM((160,), jnp.int32),
        pltpu.VMEM((16, ROWS), jnp.float32),
        pltpu.VMEM((16, ROWS), jnp.float32),
        pltpu.SemaphoreType.DMA,
        pltpu.SemaphoreType.DMA,
        pltpu.SemaphoreType.DMA,
        pltpu.SemaphoreType.DMA,
        pltpu.SemaphoreType.DMA,
        pltpu.SemaphoreType.DMA,
    ],
    compiler_params=_sc_params,
)
def _scatter_kernel(feat_hbm, pks_hbm, gst_hbm, out_hbm, acc, seg0, seg1,
                    idxg0, idxg1, rows0, rows1, gbuf, tbuf0, tbuf1, ssem0,
                    ssem1, gsem0, gsem1, tsem0, tsem1):
  w = _wid()
  lanes = jnp.arange(16, dtype=jnp.int32)
  neg = jnp.full((16,), -jnp.inf, dtype=jnp.float32)
  segs = (seg0, seg1)
  ssems = (ssem0, ssem1)
  idxgs = (idxg0, idxg1)
  rowss = (rows0, rows1)
  gsems = (gsem0, gsem1)
  tbufs = (tbuf0, tbuf1)
  tsems = (tsem0, tsem1)
  pltpu.sync_copy(gst_hbm, gbuf)

  def sissue(ach, c, par):
    pltpu.async_copy(pks_hbm.at[pl.ds(ach + c * SEGCH, SEGCH)],
                     segs[par].at[pl.ds(0, SEGCH)], ssems[par])

  def swait(ach, c, par):
    pltpu.make_async_copy(pks_hbm.at[pl.ds(ach + c * SEGCH, SEGCH)],
                          segs[par].at[pl.ds(0, SEGCH)], ssems[par]).wait()

  def gissue(spar, base, gpar):
    seg = segs[spar]
    for u in range(G // 16):
      pk = seg[pl.ds(base + u * 16, 16)]
      idxgs[gpar][pl.ds(u * 16, 16)] = jnp.minimum(pk & 0x1FFFF, N - 1)
    pltpu.async_copy(feat_hbm.at[idxgs[gpar]], rowss[gpar], gsems[gpar])

  def rmw_batch(spar, base, nj, gpar):
    pltpu.make_async_copy(feat_hbm.at[idxgs[gpar]], rowss[gpar],
                          gsems[gpar]).wait()
    seg = segs[spar]
    rows = rowss[gpar]

    def rmw(j, _):
      pk = seg[pl.ds(base + j, 16)][0]
      off = (pk >> 17) * SP
      for u in range(C // 16):
        a = acc[pl.ds(off + u * 16, 16)]
        f = rows[j, pl.ds(u * 16, 16)]
        acc[pl.ds(off + u * 16, 16)] = jnp.maximum(a, f)
      return 0

    lax.fori_loop(0, nj, rmw, 0)

  def do_chunk(spar, s_c, ne_c):
    nbat = (ne_c + G - 1) // G

    @pl.when(nbat > 0)
    def _():
      gissue(spar, s_c, 0)

      def bpair(bb, _):
        b0 = bb * 2
        base0 = s_c + b0 * G
        rmw0 = jnp.minimum(jnp.int32(G), ne_c - b0 * G)

        @pl.when(b0 + 1 < nbat)
        def _():
          gissue(spar, base0 + G, 1)

        rmw_batch(spar, base0, rmw0, 0)

        @pl.when(b0 + 1 < nbat)
        def _():
          @pl.when(b0 + 2 < nbat)
          def _():
            gissue(spar, base0 + 2 * G, 0)

          rmw_batch(spar, base0 + G,
                    jnp.minimum(jnp.int32(G), ne_c - (b0 + 1) * G), 1)

        return 0

      lax.fori_loop(0, (nbat + 1) // 2, bpair, 0)

  def do_pass(p, _):
    rid = p * NW + w
    gs = gbuf[pl.ds(rid, 16)][0]
    ge = gbuf[pl.ds(rid + 1, 16)][0]
    nseg = ge - gs
    ach = (gs // 8) * 8
    hd = gs - ach
    nch = (hd + nseg + SEGCH - 1) // SEGCH

    def initb(i, _):
      for u in range(8):
        acc[pl.ds(i * 128 + u * 16, 16)] = neg
      return 0

    lax.fori_loop(0, ROWS * SP // 128, initb, 0)

    @pl.when(nch > 0)
    def _():
      sissue(ach, 0, 0)

    def cpair(cc, _):
      c0 = cc * 2

      def one(c, par):
        swait(ach, c, par)

        @pl.when(c + 1 < nch)
        def _():
          sissue(ach, c + 1, 1 - par)

        s_c = jnp.where(c == 0, hd, 0)
        ne_c = jnp.minimum(jnp.int32(SEGCH), hd + nseg - c * SEGCH) - s_c
        do_chunk(par, s_c, ne_c)

      one(c0, 0)

      @pl.when(c0 + 1 < nch)
      def _():
        one(c0 + 1, 1)

      return 0

    lax.fori_loop(0, (nch + 1) // 2, cpair, 0)

    b_out = rid // 2
    h512 = (rid % 2) * ROWS

    def strip_dst(cg):
      return out_hbm.at[b_out, pl.ds(cg * 16, 16), pl.ds(h512, ROWS)]

    lanes_sp = lanes * SP
    for cg in range(8):
      tb = tbufs[cg % 2]
      if cg >= 2:
        pltpu.make_async_copy(tb, strip_dst(cg - 2), tsems[cg % 2]).wait()
      else:
        @pl.when(p > 0)
        def _(tb=tb, cg=cg):
          pltpu.make_async_copy(tb, strip_dst(cg), tsems[cg % 2]).wait()

      def tcol(i, _, cg=cg, tb=tb):
        j = i // 32
        rg = i - j * 32
        vec = plsc.load_gather(acc, [lanes_sp + (rg * 16 * SP + cg * 16 + j)])
        vec = jnp.where(vec == -jnp.inf, jnp.float32(0.0), vec)
        tb[j, pl.ds(rg * 16, 16)] = vec
        return 0

      lax.fori_loop(0, 16 * 32, tcol, 0)
      pltpu.async_copy(tb, strip_dst(cg), tsems[cg % 2])
    return 0

  lax.fori_loop(0, NPASS, do_pass, 0)
  final_dst = out_hbm.at[0, pl.ds(0, 16), pl.ds(0, ROWS)]
  pltpu.make_async_copy(tbufs[0], final_dst, tsems[0]).wait()
  pltpu.make_async_copy(tbufs[1], final_dst, tsems[1]).wait()


def kernel(features, coors, batch_size):
  del batch_size
  coflat = jnp.reshape(coors.astype(jnp.int32), (-1,))
  coflat = jnp.concatenate([coflat, jnp.zeros((32,), jnp.int32)])
  keys, hist = _keys_kernel(coflat)
  pks, gst = _permute_kernel(keys, hist)
  dense = _scatter_kernel(features, pks, gst)
  return dense.reshape(BATCH, C, OY, OX)


# speedup vs baseline: 1.3034x; 1.3034x over previous
import functools

import jax
import jax.numpy as jnp
from jax import lax
from jax.experimental import pallas as pl
from jax.experimental.pallas import tpu as pltpu
from jax.experimental.pallas import tpu_sc as plsc

N = 100000
C = 128
BATCH = 64
OY, OX = 32, 32
NOUT = BATCH * OY * OX
NC, NS = 2, 16
NW = NC * NS

PTS_W = N // NW
KPAD = 3136
CO_W = 3 * PTS_W
CBUF = 9392
SENT = 1 << 20
NKEYS = NW * KPAD

NREG = 128
ROWS = NOUT // NREG
SP = 129
NPASS = NREG // NW
G = 128
SEGCH = 3072
SEGBUF = SEGCH + G
PKS = 105472
HALF = 52352
NHALF = N // NC

_mesh = plsc.VectorSubcoreMesh(
    core_axis_name="c", subcore_axis_name="s", num_cores=NC, num_subcores=NS)
_sc_params = pltpu.CompilerParams(needs_layout_passes=False)


def _wid():
  return lax.axis_index("s") * NC + lax.axis_index("c")


@functools.partial(
    pl.kernel,
    out_type=[jax.ShapeDtypeStruct((NKEYS,), jnp.int32),
              jax.ShapeDtypeStruct((NW * NREG,), jnp.int32)],
    mesh=_mesh,
    scratch_types=[
        pltpu.VMEM((CBUF,), jnp.int32),
        pltpu.VMEM((KPAD,), jnp.int32),
        pltpu.VMEM((NREG * 16,), jnp.int32),
        pltpu.VMEM((NREG,), jnp.int32),
    ],
    compiler_params=_sc_params,
)
def _keys_kernel(coors_hbm, keys_hbm, hist_hbm, cbuf, kbuf, hsub, hrow):
  w = _wid()
  start = w * CO_W
  astart = (start // 8) * 8
  r = start - astart
  pltpu.sync_copy(coors_hbm.at[pl.ds(astart, CBUF)], cbuf)
  lanes = jnp.arange(16, dtype=jnp.int32)
  zeros = jnp.zeros((16,), jnp.int32)
  ones = jnp.ones((16,), jnp.int32)

  def hz(i, _):
    hsub[pl.ds(i * 16, 16)] = zeros
    return 0

  lax.fori_loop(0, NREG, hz, 0)

  def body(i, _):
    p = i * 16
    valid = (p + lanes) < PTS_W
    base = r + 3 * p + lanes * 3
    idx = jnp.where(valid, base, 0)
    bcol = plsc.load_gather(cbuf, [idx])
    ycol = plsc.load_gather(cbuf, [idx + 1])
    xcol = plsc.load_gather(cbuf, [idx + 2])
    key = bcol * (OY * OX) + (ycol >> 1) * OX + (xcol >> 1)
    kbuf[pl.ds(p, 16)] = jnp.where(valid, key, SENT)
    reg = jnp.where(valid, key >> 9, 0)
    plsc.addupdate_scatter(hsub, [reg * 16 + lanes], ones, mask=valid)
    return 0

  lax.fori_loop(0, KPAD // 16, body, 0)
  for rg in range(NREG // 16):
    acc16 = zeros
    for rr in range(16):
      s = jnp.sum(hsub[pl.ds((rg * 16 + rr) * 16, 16)], axis=0)
      acc16 = jnp.where(lanes == rr, s, acc16)
    hrow[pl.ds(rg * 16, 16)] = acc16
  pltpu.sync_copy(kbuf, keys_hbm.at[pl.ds(w * KPAD, KPAD)])
  pltpu.sync_copy(hrow, hist_hbm.at[pl.ds(w * NREG, NREG)])


@functools.partial(
    pl.kernel,
    out_type=[jax.ShapeDtypeStruct((PKS,), jnp.int32),
              jax.ShapeDtypeStruct((336,), jnp.int32)],
    mesh=_mesh,
    scratch_types=[
        pltpu.VMEM((KPAD,), jnp.int32),
        pltpu.VMEM((NW * NREG,), jnp.int32),
        pltpu.VMEM((NREG + 16,), jnp.int32),
        pltpu.VMEM((160,), jnp.int32),
        pltpu.VMEM((KPAD,), jnp.int32),
        pltpu.VMEM((3216,), jnp.int32),
        pltpu.VMEM((3216,), jnp.int32),
        pltpu.VMEM((8, G), jnp.int32),
        pltpu.VMEM((8, G), jnp.int32),
        pltpu.VMEM((2048,), jnp.int32),
        pltpu.VMEM((48,), jnp.int32),
        pltpu.VMEM_SHARED((50048,), jnp.int32),
    ] + [pltpu.SemaphoreType.DMA] * 8,
    compiler_params=_sc_params,
)
def _permute_kernel(keys_hbm, hist_hbm, pks_hbm, gst_hbm, kbuf, hbuf, offs,
                    gsb, regb, ent, dst, dstb, entb, zbuf, shbuf, shent,
                    *psems):
  w = _wid()
  myc = w & 1
  pltpu.sync_copy(keys_hbm.at[pl.ds(w * KPAD, KPAD)], kbuf)
  pltpu.sync_copy(hist_hbm, hbuf)
  lanes = jnp.arange(16, dtype=jnp.int32)
  zeros = jnp.zeros((16,), jnp.int32)
  lane0 = lanes == 0
  wv = jnp.full((16,), w, jnp.int32)

  carry_gs = jnp.int32(0)
  for rg in range(NREG // 16):
    def accw(wp, c, rg=rg):
      tot, mine = c
      row = hbuf[pl.ds(wp * NREG + rg * 16, 16)]
      samesc = jnp.full((16,), (wp & 1) == (w & 1), jnp.bool_)
      row = jnp.where(samesc, row, 0)
      tot = tot + row
      mine = mine + jnp.where(jnp.full((16,), wp, jnp.int32) < wv, row, 0)
      return (tot, mine)

    tot, mine = lax.fori_loop(0, NW, accw, (zeros, zeros))
    ex = plsc.cumsum(tot) - tot
    gs_vec = ex + carry_gs
    offs[pl.ds(rg * 16, 16)] = gs_vec + mine
    gsb[pl.ds(rg * 16, 16)] = gs_vec + myc * HALF
    carry_gs = carry_gs + jnp.sum(tot, axis=0)
  sent16 = jnp.full((16,), NHALF, jnp.int32) + myc * HALF
  gsb[pl.ds(NREG, 16)] = sent16
  gsb[pl.ds(NREG + 16, 16)] = sent16

  @pl.when(w < 2)
  def _():
    pltpu.sync_copy(gsb.at[pl.ds(0, 160)],
                    gst_hbm.at[pl.ds(w * 160, 160)])

  def zb(i, _):
    zbuf[pl.ds(i * 16, 16)] = zeros
    return 0

  lax.fori_loop(0, 128, zb, 0)

  @pl.when(w == 2)
  def _():
    pltpu.sync_copy(zbuf, pks_hbm.at[pl.ds(NHALF, 2048)])
    pltpu.sync_copy(zbuf.at[pl.ds(0, HALF - NHALF - 2048)],
                    pks_hbm.at[pl.ds(NHALF + 2048, HALF - NHALF - 2048)])

  @pl.when(w == 3)
  def _():
    pltpu.sync_copy(zbuf, pks_hbm.at[pl.ds(HALF + NHALF, 2048)])
    pltpu.sync_copy(zbuf.at[pl.ds(0, PKS - HALF - NHALF - 2048)],
                    pks_hbm.at[pl.ds(HALF + NHALF + 2048,
                                     PKS - HALF - NHALF - 2048)])

  del lane0
  def mkent(i, _):
    p = i * 16
    key = kbuf[pl.ds(p, 16)]
    ent[pl.ds(p, 16)] = ((key & 511) << 17) | (w * PTS_W + p + lanes)
    regb[pl.ds(p, 16)] = jnp.minimum(key >> 9, NREG)
    return 0

  lax.fori_loop(0, KPAD // 16, mkent, 0)

  big = jnp.full((16,), 0x7FFFFFFF, jnp.int32)
  negone = jnp.full((16,), -1, jnp.int32)

  def pj16(i, _):
    p = i * 16
    reg = regb[pl.ds(p, 16)]
    srt_reg, srt_lane = plsc.sort_key_val(reg, lanes)
    shbuf[pl.ds(0, 16)] = negone
    shbuf[pl.ds(1, 16)] = srt_reg
    prev = shbuf[pl.ds(0, 16)]
    shbuf[pl.ds(16, 16)] = big
    shbuf[pl.ds(0, 16)] = srt_reg
    nxt = shbuf[pl.ds(1, 16)]
    newrun = srt_reg != prev
    runend = srt_reg != nxt
    runstart = plsc.cummax(jnp.where(newrun, lanes, 0))
    rank = lanes - runstart
    base = plsc.load_gather(offs, [srt_reg])
    dstv = base + rank
    plsc.store_scatter(offs, [srt_reg], dstv + 1, mask=runend)
    plsc.store_scatter(dst, [srt_lane + p], dstv)
    return 0

  lax.fori_loop(0, KPAD // 16, pj16, 0)

  le = jnp.full((16,), ent[pl.ds(PTS_W - 1, 16)][0], jnp.int32)
  ld = jnp.full((16,), dst[pl.ds(PTS_W - 1, 16)][0], jnp.int32)
  for t in range(5):
    ent[pl.ds(PTS_W + t * 16, 16)] = le
    dst[pl.ds(PTS_W + t * 16, 16)] = ld

  nb = (PTS_W + G - 1) // G

  for b in range(nb):
    k = b % 8
    if b >= 8:
      pltpu.make_async_copy(entb.at[k], shent.at[dstb.at[k]],
                            psems[k]).wait()
    for u in range(G // 16):
      dstb[k, pl.ds(u * 16, 16)] = dst[pl.ds(b * G + u * 16, 16)]
      entb[k, pl.ds(u * 16, 16)] = ent[pl.ds(b * G + u * 16, 16)]
    pltpu.async_copy(entb.at[k], shent.at[dstb.at[k]], psems[k])
  for k in range(8):
    pltpu.make_async_copy(entb.at[k], shent.at[dstb.at[k]],
                          psems[k]).wait()
  plsc.subcore_barrier()

  @pl.when(lax.axis_index("s") == 0)
  def _():
    pltpu.sync_copy(shent, pks_hbm.at[pl.ds(myc * HALF, 50048)])


@functools.partial(
    pl.kernel,
    out_type=jax.ShapeDtypeStruct((BATCH, C, OY * OX), jnp.float32),
    mesh=_mesh,
    scratch_types=[
        pltpu.VMEM((ROWS * SP,), jnp.float32),
        pltpu.VMEM((SEGBUF,), jnp.int32),
        pltpu.VMEM((SEGBUF,), jnp.int32),
        pltpu.VMEM((G,), jnp.int32),
        pltpu.VMEM((G,), jnp.int32),
        pltpu.VMEM((G, C), jnp.float32),
        pltpu.VMEM((G, C), jnp.float32),
        pltpu.VMEM((336,), jnp.int32),
        pltpu.VMEM((16, ROWS), jnp.float32),
        pltpu.VMEM((16, ROWS), jnp.float32),
        pltpu.SemaphoreType.DMA,
        pltpu.SemaphoreType.DMA,
        pltpu.SemaphoreType.DMA,
        pltpu.SemaphoreType.DMA,
        pltpu.SemaphoreType.DMA,
        pltpu.SemaphoreType.DMA,
    ],
    compiler_params=_sc_params,
)
def _scatter_kernel(feat_hbm, pks_hbm, gst_hbm, out_hbm, acc, seg0, seg1,
                    idxg0, idxg1, rows0, rows1, gbuf, tbuf0, tbuf1, ssem0,
                    ssem1, gsem0, gsem1, tsem0, tsem1):
  w = _wid()
  lanes = jnp.arange(16, dtype=jnp.int32)
  neg = jnp.full((16,), -jnp.inf, dtype=jnp.float32)
  segs = (seg0, seg1)
  ssems = (ssem0, ssem1)
  idxgs = (idxg0, idxg1)
  rowss = (rows0, rows1)
  gsems = (gsem0, gsem1)
  tbufs = (tbuf0, tbuf1)
  tsems = (tsem0, tsem1)
  pltpu.sync_copy(gst_hbm, gbuf)

  def sissue(ach, c, par):
    pltpu.async_copy(pks_hbm.at[pl.ds(ach + c * SEGCH, SEGCH)],
                     segs[par].at[pl.ds(0, SEGCH)], ssems[par])

  def swait(ach, c, par):
    pltpu.make_async_copy(pks_hbm.at[pl.ds(ach + c * SEGCH, SEGCH)],
                          segs[par].at[pl.ds(0, SEGCH)], ssems[par]).wait()

  def gissue(spar, base, gpar):
    seg = segs[spar]
    for u in range(G // 16):
      pk = seg[pl.ds(base + u * 16, 16)]
      idxgs[gpar][pl.ds(u * 16, 16)] = jnp.minimum(pk & 0x1FFFF, N - 1)
    pltpu.async_copy(feat_hbm.at[idxgs[gpar]], rowss[gpar], gsems[gpar])

  def rmw_batch(spar, base, nj, gpar):
    pltpu.make_async_copy(feat_hbm.at[idxgs[gpar]], rowss[gpar],
                          gsems[gpar]).wait()
    seg = segs[spar]
    rows = rowss[gpar]

    def rmw(j, _):
      pk = seg[pl.ds(base + j, 16)][0]
      off = (pk >> 17) * SP
      for u in range(C // 16):
        a = acc[pl.ds(off + u * 16, 16)]
        f = rows[j, pl.ds(u * 16, 16)]
        acc[pl.ds(off + u * 16, 16)] = jnp.maximum(a, f)
      return 0

    lax.fori_loop(0, nj, rmw, 0)

  def do_chunk(spar, s_c, ne_c):
    nbat = (ne_c + G - 1) // G

    @pl.when(nbat > 0)
    def _():
      gissue(spar, s_c, 0)

      def bpair(bb, _):
        b0 = bb * 2
        base0 = s_c + b0 * G
        rmw0 = jnp.minimum(jnp.int32(G), ne_c - b0 * G)

        @pl.when(b0 + 1 < nbat)
        def _():
          gissue(spar, base0 + G, 1)

        rmw_batch(spar, base0, rmw0, 0)

        @pl.when(b0 + 1 < nbat)
        def _():
          @pl.when(b0 + 2 < nbat)
          def _():
            gissue(spar, base0 + 2 * G, 0)

          rmw_batch(spar, base0 + G,
                    jnp.minimum(jnp.int32(G), ne_c - (b0 + 1) * G), 1)

        return 0

      lax.fori_loop(0, (nbat + 1) // 2, bpair, 0)

  def do_segment(gs, ge):
    nseg = ge - gs
    ach = (gs // 8) * 8
    hd = gs - ach
    nch = (hd + nseg + SEGCH - 1) // SEGCH

    @pl.when(nch > 0)
    def _():
      sissue(ach, 0, 0)

    def cpair(cc, _):
      c0 = cc * 2

      def one(c, par):
        swait(ach, c, par)

        @pl.when(c + 1 < nch)
        def _():
          sissue(ach, c + 1, 1 - par)

        s_c = jnp.where(c == 0, hd, 0)
        ne_c = jnp.minimum(jnp.int32(SEGCH), hd + nseg - c * SEGCH) - s_c
        do_chunk(par, s_c, ne_c)

      one(c0, 0)

      @pl.when(c0 + 1 < nch)
      def _():
        one(c0 + 1, 1)

      return 0

    lax.fori_loop(0, (nch + 1) // 2, cpair, 0)

  def do_pass(p, _):
    rid = p * NW + w

    def initb(i, _):
      for u in range(8):
        acc[pl.ds(i * 128 + u * 16, 16)] = neg
      return 0

    lax.fori_loop(0, ROWS * SP // 128, initb, 0)

    for hf in range(2):
      gs = gbuf[pl.ds(hf * 160 + rid, 16)][0]
      ge = gbuf[pl.ds(hf * 160 + rid + 1, 16)][0]
      do_segment(gs, ge)

    b_out = rid // 2
    h512 = (rid % 2) * ROWS

    def strip_dst(cg):
      return out_hbm.at[b_out, pl.ds(cg * 16, 16), pl.ds(h512, ROWS)]

    lanes_sp = lanes * SP
    for cg in range(8):
      tb = tbufs[cg % 2]
      if cg >= 2:
        pltpu.make_async_copy(tb, strip_dst(cg - 2), tsems[cg % 2]).wait()
      else:
        @pl.when(p > 0)
        def _(tb=tb, cg=cg):
          pltpu.make_async_copy(tb, strip_dst(cg), tsems[cg % 2]).wait()

      def tcol(i, _, cg=cg, tb=tb):
        j = i // 32
        rg = i - j * 32
        vec = plsc.load_gather(acc, [lanes_sp + (rg * 16 * SP + cg * 16 + j)])
        vec = jnp.where(vec == -jnp.inf, jnp.float32(0.0), vec)
        tb[j, pl.ds(rg * 16, 16)] = vec
        return 0

      lax.fori_loop(0, 16 * 32, tcol, 0)
      pltpu.async_copy(tb, strip_dst(cg), tsems[cg % 2])
    return 0

  lax.fori_loop(0, NPASS, do_pass, 0)
  final_dst = out_hbm.at[0, pl.ds(0, 16), pl.ds(0, ROWS)]
  pltpu.make_async_copy(tbufs[0], final_dst, tsems[0]).wait()
  pltpu.make_async_copy(tbufs[1], final_dst, tsems[1]).wait()


def kernel(features, coors, batch_size):
  del batch_size
  coflat = jnp.reshape(coors.astype(jnp.int32), (-1,))
  coflat = jnp.concatenate([coflat, jnp.zeros((32,), jnp.int32)])
  keys, hist = _keys_kernel(coflat)
  pks, gst = _permute_kernel(keys, hist)
  dense = _scatter_kernel(features, pks, gst)
  return dense.reshape(BATCH, C, OY, OX)
